# baseline (speedup 1.0000x reference)
import functools

import jax
import jax.numpy as jnp
from jax import lax
from jax.experimental import pallas as pl
from jax.experimental.pallas import tpu as pltpu
from jax.experimental.pallas import tpu_sc as plsc

B = 16
L = 4096
PERIOD = 24
HALF = 12
CHUNK = 2048
NVEC = L // 16

INV_C0 = 1.0 / 171.0
INV_C1 = 1.0 / 170.0


def _body(x_hbm, out_hbm, xl, csl, tbuf, dbuf, ebuf, pmbuf, bins, obuf):
    core = lax.axis_index("c")
    sub = lax.axis_index("s")
    row = core * 8 + sub // 2
    h = sub % 2
    st = h * CHUNK

    iota = lax.iota(jnp.int32, 16)
    zero = jnp.zeros((16,), jnp.float32)

    pltpu.sync_copy(x_hbm.at[pl.ds(row * L, L)], xl)

    bins[pl.ds(0, 16)] = zero
    bins[pl.ds(16, 16)] = zero

    stride_idx = iota * 16

    def grp_body(g, carry):
        s = plsc.load_gather(xl, [256 * g + stride_idx])
        for j in range(1, 16):
            s = s + plsc.load_gather(xl, [256 * g + j + stride_idx])
        ebuf[pl.ds(16 * g, 16)] = plsc.cumsum(s) - s + carry
        return carry + jnp.sum(s)

    lax.fori_loop(0, NVEC // 16, grp_body, jnp.float32(0.0))

    @plsc.parallel_loop(0, NVEC)
    def cs_body(k):
        base = plsc.load_gather(ebuf, [jnp.broadcast_to(k, (16,))])
        csl[pl.ds(16 * k, 16)] = plsc.cumsum(xl[pl.ds(16 * k, 16)]) + base

    @plsc.parallel_loop(0, NVEC)
    def t_body(k):
        gi = 16 * k + iota
        end_g = jnp.minimum(gi + (HALF + 1), L)
        start_g = jnp.maximum(gi - HALF, 0)
        cnt = (end_g - start_g).astype(jnp.float32)
        sum_e = plsc.load_gather(csl, [end_g - 1])
        s_loc = start_g - 1
        sum_s = plsc.load_gather(csl, [jnp.maximum(s_loc, 0)])
        sum_s = jnp.where(s_loc >= 0, sum_s, 0.0)
        t = (sum_e - sum_s) / cnt
        d = xl[pl.ds(16 * k, 16)] - t
        tbuf[pl.ds(16 * k, 16)] = t
        dbuf[pl.ds(16 * k, 16)] = d
        plsc.addupdate_scatter(bins, [lax.rem(gi, PERIOD)], d)

    pmbuf[pl.ds(0, 16)] = bins[pl.ds(0, 16)] * INV_C0
    pmbuf[pl.ds(16, 16)] = bins[pl.ds(16, 16)] * INV_C1

    @plsc.parallel_loop(0, CHUNK // 16)
    def o_body(k):
        gi = st + 16 * k + iota
        t = tbuf[pl.ds(st + 16 * k, 16)]
        d = dbuf[pl.ds(st + 16 * k, 16)]
        sv = plsc.load_gather(pmbuf, [lax.rem(gi, PERIOD)])
        o3 = (16 * k + iota) * 3
        plsc.store_scatter(obuf, [o3], t)
        plsc.store_scatter(obuf, [o3 + 1], sv)
        plsc.store_scatter(obuf, [o3 + 2], d - sv)

    pltpu.sync_copy(obuf, out_hbm.at[pl.ds((row * L + st) * 3, CHUNK * 3)])


_decomp_sc = functools.partial(
    pl.kernel,
    mesh=plsc.VectorSubcoreMesh(core_axis_name="c", subcore_axis_name="s"),
    out_type=jax.ShapeDtypeStruct((B * L * 3,), jnp.float32),
    compiler_params=pltpu.CompilerParams(needs_layout_passes=False),
    scratch_types=[
        pltpu.VMEM((L,), jnp.float32),
        pltpu.VMEM((L,), jnp.float32),
        pltpu.VMEM((L,), jnp.float32),
        pltpu.VMEM((L,), jnp.float32),
        pltpu.VMEM((NVEC,), jnp.float32),
        pltpu.VMEM((32,), jnp.float32),
        pltpu.VMEM((32,), jnp.float32),
        pltpu.VMEM((CHUNK * 3,), jnp.float32),
    ],
)(_body)


@jax.jit
def kernel(inputs):
    out = _decomp_sc(inputs.reshape(-1))
    return out.reshape(B, L, 3)


# speedup vs baseline: 1.0186x; 1.0186x over previous
import functools

import jax
import jax.numpy as jnp
from jax import lax
from jax.experimental import pallas as pl
from jax.experimental.pallas import tpu as pltpu
from jax.experimental.pallas import tpu_sc as plsc

B = 16
L = 4096
PERIOD = 24
HALF = 12
CHUNK = 2048
NVEC = L // 16

INV_C0 = 1.0 / 171.0
INV_C1 = 1.0 / 170.0


def _body(x_hbm, out_hbm, xl, csl, tbuf, dbuf, pmbuf, bins, obuf):
    core = lax.axis_index("c")
    sub = lax.axis_index("s")
    row = core * 8 + sub // 2
    h = sub % 2
    st = h * CHUNK

    iota = lax.iota(jnp.int32, 16)
    zero = jnp.zeros((16,), jnp.float32)

    pltpu.sync_copy(x_hbm.at[pl.ds(row * L, L)], xl)

    bins[pl.ds(0, 16)] = zero
    bins[pl.ds(16, 16)] = zero

    @plsc.parallel_loop(0, NVEC, carry=jnp.float32(0.0))
    def cs_body(k, carry):
        chunk = xl[pl.ds(16 * k, 16)]
        csl[pl.ds(16 * k, 16)] = plsc.cumsum(chunk) + carry
        return carry + jnp.sum(chunk)

    @plsc.parallel_loop(0, NVEC)
    def t_body(k):
        gi = 16 * k + iota
        end_g = jnp.minimum(gi + (HALF + 1), L)
        start_g = jnp.maximum(gi - HALF, 0)
        cnt = (end_g - start_g).astype(jnp.float32)
        sum_e = plsc.load_gather(csl, [end_g - 1])
        s_loc = start_g - 1
        sum_s = plsc.load_gather(csl, [jnp.maximum(s_loc, 0)])
        sum_s = jnp.where(s_loc >= 0, sum_s, 0.0)
        t = (sum_e - sum_s) / cnt
        d = xl[pl.ds(16 * k, 16)] - t
        tbuf[pl.ds(16 * k, 16)] = t
        dbuf[pl.ds(16 * k, 16)] = d
        plsc.addupdate_scatter(bins, [lax.rem(gi, PERIOD)], d)

    pmbuf[pl.ds(0, 16)] = bins[pl.ds(0, 16)] * INV_C0
    pmbuf[pl.ds(16, 16)] = bins[pl.ds(16, 16)] * INV_C1

    @plsc.parallel_loop(0, CHUNK // 16)
    def o_body(k):
        gi = st + 16 * k + iota
        t = tbuf[pl.ds(st + 16 * k, 16)]
        d = dbuf[pl.ds(st + 16 * k, 16)]
        sv = plsc.load_gather(pmbuf, [lax.rem(gi, PERIOD)])
        o3 = (16 * k + iota) * 3
        plsc.store_scatter(obuf, [o3], t)
        plsc.store_scatter(obuf, [o3 + 1], sv)
        plsc.store_scatter(obuf, [o3 + 2], d - sv)

    pltpu.sync_copy(obuf, out_hbm.at[pl.ds((row * L + st) * 3, CHUNK * 3)])


_decomp_sc = functools.partial(
    pl.kernel,
    mesh=plsc.VectorSubcoreMesh(core_axis_name="c", subcore_axis_name="s"),
    out_type=jax.ShapeDtypeStruct((B * L * 3,), jnp.float32),
    compiler_params=pltpu.CompilerParams(needs_layout_passes=False),
    scratch_types=[
        pltpu.VMEM((L,), jnp.float32),
        pltpu.VMEM((L,), jnp.float32),
        pltpu.VMEM((L,), jnp.float32),
        pltpu.VMEM((L,), jnp.float32),
        pltpu.VMEM((32,), jnp.float32),
        pltpu.VMEM((32,), jnp.float32),
        pltpu.VMEM((CHUNK * 3,), jnp.float32),
    ],
)(_body)


@jax.jit
def kernel(inputs):
    out = _decomp_sc(inputs.reshape(-1))
    return out.reshape(B, L, 3)


# speedup vs baseline: 1.0206x; 1.0020x over previous
import functools

import jax
import jax.numpy as jnp
from jax import lax
from jax.experimental import pallas as pl
from jax.experimental.pallas import tpu as pltpu
from jax.experimental.pallas import tpu_sc as plsc

B = 16
L = 4096
PERIOD = 24
HALF = 12
CHUNK = 2048
NVEC = L // 16
INV_W = 1.0 / (2 * HALF + 1)

INV_C0 = 1.0 / 171.0
INV_C1 = 1.0 / 170.0


def _body(x_hbm, out_hbm, xl, cslp, tbuf, dbuf, pmbuf, bins, obuf):
    core = lax.axis_index("c")
    sub = lax.axis_index("s")
    row = core * 8 + sub // 2
    h = sub % 2
    st = h * CHUNK

    iota = lax.iota(jnp.int32, 16)
    zero = jnp.zeros((16,), jnp.float32)

    pltpu.sync_copy(x_hbm.at[pl.ds(row * L, L)], xl)

    bins[pl.ds(0, 16)] = zero
    bins[pl.ds(16, 16)] = zero
    cslp[pl.ds(0, 16)] = zero

    @plsc.parallel_loop(0, NVEC, carry=jnp.float32(0.0))
    def cs_body(k, carry):
        chunk = xl[pl.ds(16 * k, 16)]
        cslp[pl.ds(16 + 16 * k, 16)] = plsc.cumsum(chunk) + carry
        return carry + jnp.sum(chunk)

    cslp[pl.ds(16 + L, 16)] = jnp.broadcast_to(cs_body, (16,))

    @plsc.parallel_loop(1, NVEC - 1)
    def t_body(k):
        gi = 16 * k + iota
        t = (plsc.load_gather(cslp, [gi + 28])
             - plsc.load_gather(cslp, [gi + 3])) * INV_W
        d = xl[pl.ds(16 * k, 16)] - t
        tbuf[pl.ds(16 * k, 16)] = t
        dbuf[pl.ds(16 * k, 16)] = d
        plsc.addupdate_scatter(bins, [lax.rem(gi, PERIOD)], d)

    for k in (0, NVEC - 1):
        gi = 16 * k + iota
        cnt = (jnp.minimum(gi + (HALF + 1), L)
               - jnp.maximum(gi - HALF, 0)).astype(jnp.float32)
        t = (plsc.load_gather(cslp, [gi + 28])
             - plsc.load_gather(cslp, [gi + 3])) / cnt
        d = xl[pl.ds(16 * k, 16)] - t
        tbuf[pl.ds(16 * k, 16)] = t
        dbuf[pl.ds(16 * k, 16)] = d
        plsc.addupdate_scatter(bins, [lax.rem(gi, PERIOD)], d)

    pmbuf[pl.ds(0, 16)] = bins[pl.ds(0, 16)] * INV_C0
    pmbuf[pl.ds(16, 16)] = bins[pl.ds(16, 16)] * INV_C1

    @plsc.parallel_loop(0, CHUNK // 16)
    def o_body(k):
        gi = st + 16 * k + iota
        t = tbuf[pl.ds(st + 16 * k, 16)]
        d = dbuf[pl.ds(st + 16 * k, 16)]
        sv = plsc.load_gather(pmbuf, [lax.rem(gi, PERIOD)])
        o3 = (16 * k + iota) * 3
        plsc.store_scatter(obuf, [o3], t)
        plsc.store_scatter(obuf, [o3 + 1], sv)
        plsc.store_scatter(obuf, [o3 + 2], d - sv)

    pltpu.sync_copy(obuf, out_hbm.at[pl.ds((row * L + st) * 3, CHUNK * 3)])


_decomp_sc = functools.partial(
    pl.kernel,
    mesh=plsc.VectorSubcoreMesh(core_axis_name="c", subcore_axis_name="s"),
    out_type=jax.ShapeDtypeStruct((B * L * 3,), jnp.float32),
    compiler_params=pltpu.CompilerParams(needs_layout_passes=False),
    scratch_types=[
        pltpu.VMEM((L,), jnp.float32),
        pltpu.VMEM((L + 32,), jnp.float32),
        pltpu.VMEM((L,), jnp.float32),
        pltpu.VMEM((L,), jnp.float32),
        pltpu.VMEM((32,), jnp.float32),
        pltpu.VMEM((32,), jnp.float32),
        pltpu.VMEM((CHUNK * 3,), jnp.float32),
    ],
)(_body)


@jax.jit
def kernel(inputs):
    out = _decomp_sc(inputs.reshape(-1))
    return out.reshape(B, L, 3)
